# baseline (speedup 1.0000x reference)
import functools

import jax
import jax.numpy as jnp
from jax.experimental import pallas as pl
from jax.experimental.pallas import tpu as pltpu


def _copy_body(wpe_ref, o_ref, sem):
    copy = pltpu.make_async_copy(wpe_ref, o_ref, sem)
    copy.start()
    copy.wait()


@functools.cache
def _tc_copy(seq_len: int, emb_dim: int):
    return pl.pallas_call(
        _copy_body,
        out_shape=jax.ShapeDtypeStruct((seq_len, emb_dim), jnp.float32),
        in_specs=[pl.BlockSpec(memory_space=pl.ANY)],
        out_specs=pl.BlockSpec(memory_space=pl.ANY),
        scratch_shapes=[pltpu.SemaphoreType.DMA],
    )


def kernel(x, wpe):
    seq_len = x.shape[1]
    emb_dim = wpe.shape[1]
    out = _tc_copy(seq_len, emb_dim)(wpe[:seq_len])
    return jnp.reshape(out, (1, seq_len, emb_dim))


# speedup vs baseline: 2.7193x; 2.7193x over previous
import functools

import jax
import jax.numpy as jnp
from jax.experimental import pallas as pl


def _copy_body(wpe_ref, o_ref):
    o_ref[0] = wpe_ref[...]


@functools.cache
def _tc_copy(seq_len: int, emb_dim: int):
    return pl.pallas_call(
        _copy_body,
        out_shape=jax.ShapeDtypeStruct((1, seq_len, emb_dim), jnp.float32),
    )


def kernel(x, wpe):
    seq_len = x.shape[1]
    emb_dim = wpe.shape[1]
    return _tc_copy(seq_len, emb_dim)(wpe[:seq_len])
